# baseline (speedup 1.0000x reference)
import jax
import jax.numpy as jnp
from jax.experimental import pallas as pl


def _add_kernel(x_ref, t_ref, o_ref):
    i = pl.program_id(1)
    BL = x_ref.shape[1]
    o_ref[...] = x_ref[...] + t_ref[pl.ds(i * BL, BL), :]


def kernel(x, table):
    B, L, D = x.shape
    BB = 2
    BL = 512
    return pl.pallas_call(
        _add_kernel,
        grid=(B // BB, L // BL),
        in_specs=[
            pl.BlockSpec((BB, BL, D), lambda b, i: (b, i, 0)),
            pl.BlockSpec((L, D), lambda b, i: (0, 0)),
        ],
        out_specs=pl.BlockSpec((BB, BL, D), lambda b, i: (b, i, 0)),
        out_shape=jax.ShapeDtypeStruct(x.shape, x.dtype),
    )(x, table)


# speedup vs baseline: 1.0572x; 1.0572x over previous
import jax
import jax.numpy as jnp
from jax.experimental import pallas as pl


def _add_kernel(x_ref, t_ref, o_ref):
    o_ref[...] = x_ref[...] + t_ref[...]


def kernel(x, table):
    B, L, D = x.shape
    return pl.pallas_call(
        _add_kernel,
        grid=(B,),
        in_specs=[
            pl.BlockSpec((1, L, D), lambda b: (b, 0, 0)),
            pl.BlockSpec((L, D), lambda b: (0, 0)),
        ],
        out_specs=pl.BlockSpec((1, L, D), lambda b: (b, 0, 0)),
        out_shape=jax.ShapeDtypeStruct(x.shape, x.dtype),
    )(x, table)
